# Initial kernel scaffold; baseline (speedup 1.0000x reference)
import jax
import jax.numpy as jnp
from jax import lax
from jax.experimental import pallas as pl
from jax.experimental.pallas import tpu as pltpu
from jax.experimental.pallas import tpu_sc as plsc

_N = 100000
_E = 3200000
_NF = 16
_EF = 16
_H = 64

_NC = 2
_NS = 16
_NW = _NC * _NS
_NP = 102400
_TPW = _NP // _NS

_IB = 128
_K = 8
_CH = _IB * _K
_SB = _E // _CH
_R2 = _E // _IB

_mesh = plsc.VectorSubcoreMesh(
    core_axis_name="c", subcore_axis_name="s", num_cores=_NC, num_subcores=_NS
)


def _gather_body(x_hbm, row2_hbm, col2_hbm, xr_hbm, xc_hbm,
                 idxr_v, idxc_v, xr_v, xc_v, sem):
    c = lax.axis_index("c")
    s = lax.axis_index("s")
    wid = s * _NC + c
    nb = (_SB - wid + _NW - 1) // _NW

    def step(t, carry):
        b = wid + t * _NW
        eb = b * _CH
        pltpu.sync_copy(row2_hbm.at[pl.ds(b * _K, _K), :], idxr_v)
        pltpu.sync_copy(col2_hbm.at[pl.ds(b * _K, _K), :], idxc_v)
        descs = []
        for j in range(_K):
            descs.append(pltpu.async_copy(
                x_hbm.at[idxr_v.at[j]], xr_v.at[pl.ds(j * _IB, _IB), :], sem))
            descs.append(pltpu.async_copy(
                x_hbm.at[idxc_v.at[j]], xc_v.at[pl.ds(j * _IB, _IB), :], sem))
        for d in descs:
            d.wait()
        pltpu.sync_copy(xr_v, xr_hbm.at[pl.ds(eb, _CH), :])
        pltpu.sync_copy(xc_v, xc_hbm.at[pl.ds(eb, _CH), :])
        return carry

    lax.fori_loop(0, nb, step, 0)


_gather_call = pl.kernel(
    _gather_body,
    out_type=(
        jax.ShapeDtypeStruct((_E, _NF), jnp.float32),
        jax.ShapeDtypeStruct((_E, _NF), jnp.float32),
    ),
    mesh=_mesh,
    scratch_types=[
        pltpu.VMEM((_K, _IB), jnp.int32),
        pltpu.VMEM((_K, _IB), jnp.int32),
        pltpu.VMEM((_CH, _NF), jnp.float32),
        pltpu.VMEM((_CH, _NF), jnp.float32),
        pltpu.SemaphoreType.DMA,
    ],
)


def _scatter_body(row2_hbm, emb_hbm, accp_hbm, degp_hbm,
                  idxr_v, emb_v, ones_v, zrow_v, zdeg_v, acc_sh, deg_sh):
    c = lax.axis_index("c")
    s = lax.axis_index("s")
    wid = s * _NC + c

    def zfill_row(i, carry):
        zrow_v[i] = jnp.zeros((16,), jnp.float32)
        return carry

    lax.fori_loop(0, 400, zfill_row, 0)
    for i in range(25):
        zdeg_v[pl.ds(i * 16, 16)] = jnp.zeros((16,), jnp.float32)
    for i in range(_IB // 16):
        ones_v[pl.ds(i * 16, 16)] = jnp.ones((16,), jnp.float32)

    def zero_sh(z, carry):
        off = s * _TPW + z * 400
        pltpu.sync_copy(zrow_v, acc_sh.at[pl.ds(off, 400), :])
        pltpu.sync_copy(zdeg_v, deg_sh.at[pl.ds(off, 400)])
        return carry

    lax.fori_loop(0, _TPW // 400, zero_sh, 0)
    plsc.subcore_barrier()

    def step(t, carry):
        b = wid + t * _NW
        eb = b * _CH
        pltpu.sync_copy(row2_hbm.at[pl.ds(b * _K, _K), :], idxr_v)
        pltpu.sync_copy(emb_hbm.at[pl.ds(eb, _CH), :], emb_v)
        for j in range(_K):
            pltpu.sync_copy(emb_v.at[pl.ds(j * _IB, _IB), :],
                            acc_sh.at[idxr_v.at[j]], add=True)
            pltpu.sync_copy(ones_v, deg_sh.at[idxr_v.at[j]], add=True)
        return carry

    nb = (_SB - wid + _NW - 1) // _NW
    lax.fori_loop(0, nb, step, 0)
    plsc.subcore_barrier()

    pltpu.sync_copy(acc_sh.at[pl.ds(s * _TPW, _TPW), :],
                    accp_hbm.at[c, pl.ds(s * _TPW, _TPW), :])
    pltpu.sync_copy(deg_sh.at[pl.ds(s * _TPW, _TPW)],
                    degp_hbm.at[c, pl.ds(s * _TPW, _TPW)])


_scatter_call = pl.kernel(
    _scatter_body,
    out_type=(
        jax.ShapeDtypeStruct((_NC, _NP, _EF), jnp.float32),
        jax.ShapeDtypeStruct((_NC, _NP), jnp.float32),
    ),
    mesh=_mesh,
    scratch_types=[
        pltpu.VMEM((_K, _IB), jnp.int32),
        pltpu.VMEM((_CH, _EF), jnp.float32),
        pltpu.VMEM((_IB,), jnp.float32),
        pltpu.VMEM((400, 16), jnp.float32),
        pltpu.VMEM((400,), jnp.float32),
        pltpu.VMEM_SHARED((_NP, _EF), jnp.float32),
        pltpu.VMEM_SHARED((_NP,), jnp.float32),
    ],
)


_BE = 12800


def _edge_mlp_body(xr_ref, xc_ref, ea_ref, w1a_ref, w1b_ref, w1c_ref,
                   b1_ref, w2_ref, b2_ref, out_ref):
    h = (jnp.dot(xr_ref[...], w1a_ref[...], preferred_element_type=jnp.float32)
         + jnp.dot(xc_ref[...], w1b_ref[...], preferred_element_type=jnp.float32)
         + jnp.dot(ea_ref[...], w1c_ref[...], preferred_element_type=jnp.float32)
         + b1_ref[...])
    h = jnp.maximum(h, 0.0)
    out_ref[...] = (jnp.dot(h, w2_ref[...], preferred_element_type=jnp.float32)
                    + b2_ref[...])


_BN = 5000


def _node_mlp_body(x_ref, p0_ref, p1_ref, d0_ref, d1_ref,
                   w1a_ref, w1b_ref, b1_ref, w2_ref, b2_ref, out_ref):
    deg = d0_ref[...] + d1_ref[...]
    inv = 1.0 / jnp.maximum(deg, 1.0)
    agg = (p0_ref[...] + p1_ref[...]) * inv
    h = (jnp.dot(x_ref[...], w1a_ref[...], preferred_element_type=jnp.float32)
         + jnp.dot(agg, w1b_ref[...], preferred_element_type=jnp.float32)
         + b1_ref[...])
    h = jnp.maximum(h, 0.0)
    out_ref[...] = (jnp.dot(h, w2_ref[...], preferred_element_type=jnp.float32)
                    + b2_ref[...])


def _rep(shape):
    return pl.BlockSpec(shape, lambda i: tuple(0 for _ in shape))


def kernel(x, edge_index, edge_attr, We1, be1, We2, be2, Wn1, bn1, Wn2, bn2):
    x = x.astype(jnp.float32)
    edge_attr = edge_attr.astype(jnp.float32)
    row2 = edge_index[0].reshape(_R2, _IB)
    col2 = edge_index[1].reshape(_R2, _IB)

    xr, xc = _gather_call(x, row2, col2)

    edge_embedding = pl.pallas_call(
        _edge_mlp_body,
        grid=(_E // _BE,),
        in_specs=[
            pl.BlockSpec((_BE, _NF), lambda i: (i, 0)),
            pl.BlockSpec((_BE, _NF), lambda i: (i, 0)),
            pl.BlockSpec((_BE, _EF), lambda i: (i, 0)),
            _rep((_NF, _H)),
            _rep((_NF, _H)),
            _rep((_EF, _H)),
            _rep((1, _H)),
            _rep((_H, _EF)),
            _rep((1, _EF)),
        ],
        out_specs=pl.BlockSpec((_BE, _EF), lambda i: (i, 0)),
        out_shape=jax.ShapeDtypeStruct((_E, _EF), jnp.float32),
    )(xr, xc, edge_attr,
      We1[0:_NF], We1[_NF:2 * _NF], We1[2 * _NF:],
      be1.reshape(1, _H), We2, be2.reshape(1, _EF))

    accp, degp = _scatter_call(row2, edge_embedding)

    node_embeddings = pl.pallas_call(
        _node_mlp_body,
        grid=(_N // _BN,),
        in_specs=[
            pl.BlockSpec((_BN, _NF), lambda i: (i, 0)),
            pl.BlockSpec((_BN, _EF), lambda i: (i, 0)),
            pl.BlockSpec((_BN, _EF), lambda i: (i, 0)),
            pl.BlockSpec((_BN, 1), lambda i: (i, 0)),
            pl.BlockSpec((_BN, 1), lambda i: (i, 0)),
            _rep((_NF, _H)),
            _rep((_EF, _H)),
            _rep((1, _H)),
            _rep((_H, _NF)),
            _rep((1, _NF)),
        ],
        out_specs=pl.BlockSpec((_BN, _NF), lambda i: (i, 0)),
        out_shape=jax.ShapeDtypeStruct((_N, _NF), jnp.float32),
    )(x, accp[0], accp[1],
      degp[0].reshape(_NP, 1), degp[1].reshape(_NP, 1),
      Wn1[0:_NF], Wn1[_NF:], bn1.reshape(1, _H), Wn2, bn2.reshape(1, _NF))

    return (edge_embedding, node_embeddings)


# speedup vs baseline: 5.3108x; 5.3108x over previous
import jax
import jax.numpy as jnp
from jax import lax
from jax.experimental import pallas as pl
from jax.experimental.pallas import tpu as pltpu
from jax.experimental.pallas import tpu_sc as plsc

_N = 100000
_E = 3200000
_NF = 16
_EF = 16
_H = 64

_NC = 2
_NS = 16
_NW = _NC * _NS
_NP = 100096
_TPW = _NP // _NS

_IB = 128
_K = 8
_CH = _IB * _K
_SB = _E // _CH
_R2 = _E // _IB

_mesh = plsc.VectorSubcoreMesh(
    core_axis_name="c", subcore_axis_name="s", num_cores=_NC, num_subcores=_NS
)


def _gather_body(x_hbm, row2_hbm, col2_hbm, xr_hbm, xc_hbm,
                 idxr_v, idxc_v, xr_v, xc_v, sem):
    c = lax.axis_index("c")
    s = lax.axis_index("s")
    wid = s * _NC + c
    nb = (_SB - wid + _NW - 1) // _NW

    def step(t, carry):
        b = wid + t * _NW
        eb = b * _CH
        pltpu.sync_copy(row2_hbm.at[pl.ds(b * _K, _K), :], idxr_v)
        pltpu.sync_copy(col2_hbm.at[pl.ds(b * _K, _K), :], idxc_v)
        descs = []
        for j in range(_K):
            descs.append(pltpu.async_copy(
                x_hbm.at[idxr_v.at[j]], xr_v.at[pl.ds(j * _IB, _IB), :], sem))
            descs.append(pltpu.async_copy(
                x_hbm.at[idxc_v.at[j]], xc_v.at[pl.ds(j * _IB, _IB), :], sem))
        for d in descs:
            d.wait()
        pltpu.sync_copy(xr_v, xr_hbm.at[pl.ds(eb, _CH), :])
        pltpu.sync_copy(xc_v, xc_hbm.at[pl.ds(eb, _CH), :])
        return carry

    lax.fori_loop(0, nb, step, 0)


_gather_call = pl.kernel(
    _gather_body,
    out_type=(
        jax.ShapeDtypeStruct((_E, _NF), jnp.float32),
        jax.ShapeDtypeStruct((_E, _NF), jnp.float32),
    ),
    mesh=_mesh,
    scratch_types=[
        pltpu.VMEM((_K, _IB), jnp.int32),
        pltpu.VMEM((_K, _IB), jnp.int32),
        pltpu.VMEM((_CH, _NF), jnp.float32),
        pltpu.VMEM((_CH, _NF), jnp.float32),
        pltpu.SemaphoreType.DMA,
    ],
    compiler_params=pltpu.CompilerParams(use_tc_tiling_on_sc=False),
)


def _scatter_body(row2_hbm, emb_hbm, accp_hbm, degp_hbm,
                  idxr_v, emb_v, ones_v, zrow_v, zdeg_v, acc_sh, deg_sh):
    c = lax.axis_index("c")
    s = lax.axis_index("s")
    wid = s * _NC + c

    def zfill_row(i, carry):
        zrow_v[i] = jnp.zeros((16,), jnp.float32)
        return carry

    lax.fori_loop(0, 368, zfill_row, 0)

    def zfill_deg(i, carry):
        zdeg_v[pl.ds(i * 16, 16)] = jnp.zeros((16,), jnp.float32)
        return carry

    lax.fori_loop(0, 368 // 16, zfill_deg, 0)
    for i in range(_IB // 16):
        ones_v[pl.ds(i * 16, 16)] = jnp.ones((16,), jnp.float32)

    def zero_sh(z, carry):
        off = s * _TPW + z * 368
        pltpu.sync_copy(zrow_v, acc_sh.at[pl.ds(off, 368), :])
        pltpu.sync_copy(zdeg_v, deg_sh.at[pl.ds(off, 368)])
        return carry

    lax.fori_loop(0, _TPW // 368, zero_sh, 0)
    plsc.subcore_barrier()

    def step(t, carry):
        b = wid + t * _NW
        eb = b * _CH
        pltpu.sync_copy(row2_hbm.at[pl.ds(b * _K, _K), :], idxr_v)
        pltpu.sync_copy(emb_hbm.at[pl.ds(eb, _CH), :], emb_v)
        for j in range(_K):
            pltpu.sync_copy(emb_v.at[pl.ds(j * _IB, _IB), :],
                            acc_sh.at[idxr_v.at[j]], add=True)
            pltpu.sync_copy(ones_v, deg_sh.at[idxr_v.at[j]], add=True)
        return carry

    nb = (_SB - wid + _NW - 1) // _NW
    lax.fori_loop(0, nb, step, 0)
    plsc.subcore_barrier()

    pltpu.sync_copy(acc_sh.at[pl.ds(s * _TPW, _TPW), :],
                    accp_hbm.at[c, pl.ds(s * _TPW, _TPW), :])
    pltpu.sync_copy(deg_sh.at[pl.ds(s * _TPW, _TPW)],
                    degp_hbm.at[c, pl.ds(s * _TPW, _TPW)])


_scatter_call = pl.kernel(
    _scatter_body,
    out_type=(
        jax.ShapeDtypeStruct((_NC, _NP, _EF), jnp.float32),
        jax.ShapeDtypeStruct((_NC, _NP), jnp.float32),
    ),
    mesh=_mesh,
    scratch_types=[
        pltpu.VMEM((_K, _IB), jnp.int32),
        pltpu.VMEM((_CH, _EF), jnp.float32),
        pltpu.VMEM((_IB,), jnp.float32),
        pltpu.VMEM((368, 16), jnp.float32),
        pltpu.VMEM((368,), jnp.float32),
        pltpu.VMEM_SHARED((_NP, _EF), jnp.float32),
        pltpu.VMEM_SHARED((_NP,), jnp.float32),
    ],
    compiler_params=pltpu.CompilerParams(use_tc_tiling_on_sc=False),
)


_BE = 12800


def _edge_mlp_body(xr_ref, xc_ref, ea_ref, w1a_ref, w1b_ref, w1c_ref,
                   b1_ref, w2_ref, b2_ref, out_ref):
    h = (jnp.dot(xr_ref[...], w1a_ref[...], preferred_element_type=jnp.float32)
         + jnp.dot(xc_ref[...], w1b_ref[...], preferred_element_type=jnp.float32)
         + jnp.dot(ea_ref[...], w1c_ref[...], preferred_element_type=jnp.float32)
         + b1_ref[...])
    h = jnp.maximum(h, 0.0)
    out_ref[...] = (jnp.dot(h, w2_ref[...], preferred_element_type=jnp.float32)
                    + b2_ref[...])


_BN = 5000


def _node_mlp_body(x_ref, p0_ref, p1_ref, d0_ref, d1_ref,
                   w1a_ref, w1b_ref, b1_ref, w2_ref, b2_ref, out_ref):
    deg = d0_ref[...] + d1_ref[...]
    inv = 1.0 / jnp.maximum(deg, 1.0)
    agg = (p0_ref[...] + p1_ref[...]) * inv
    h = (jnp.dot(x_ref[...], w1a_ref[...], preferred_element_type=jnp.float32)
         + jnp.dot(agg, w1b_ref[...], preferred_element_type=jnp.float32)
         + b1_ref[...])
    h = jnp.maximum(h, 0.0)
    out_ref[...] = (jnp.dot(h, w2_ref[...], preferred_element_type=jnp.float32)
                    + b2_ref[...])


def _rep(shape):
    return pl.BlockSpec(shape, lambda i: tuple(0 for _ in shape))


def kernel(x, edge_index, edge_attr, We1, be1, We2, be2, Wn1, bn1, Wn2, bn2):
    x = x.astype(jnp.float32)
    edge_attr = edge_attr.astype(jnp.float32)
    row2 = edge_index[0].reshape(_R2, _IB)
    col2 = edge_index[1].reshape(_R2, _IB)

    xr, xc = _gather_call(x, row2, col2)

    edge_embedding = pl.pallas_call(
        _edge_mlp_body,
        grid=(_E // _BE,),
        in_specs=[
            pl.BlockSpec((_BE, _NF), lambda i: (i, 0)),
            pl.BlockSpec((_BE, _NF), lambda i: (i, 0)),
            pl.BlockSpec((_BE, _EF), lambda i: (i, 0)),
            _rep((_NF, _H)),
            _rep((_NF, _H)),
            _rep((_EF, _H)),
            _rep((1, _H)),
            _rep((_H, _EF)),
            _rep((1, _EF)),
        ],
        out_specs=pl.BlockSpec((_BE, _EF), lambda i: (i, 0)),
        out_shape=jax.ShapeDtypeStruct((_E, _EF), jnp.float32),
    )(xr, xc, edge_attr,
      We1[0:_NF], We1[_NF:2 * _NF], We1[2 * _NF:],
      be1.reshape(1, _H), We2, be2.reshape(1, _EF))

    accp, degp = _scatter_call(row2, edge_embedding)

    node_embeddings = pl.pallas_call(
        _node_mlp_body,
        grid=(_N // _BN,),
        in_specs=[
            pl.BlockSpec((_BN, _NF), lambda i: (i, 0)),
            pl.BlockSpec((_BN, _EF), lambda i: (i, 0)),
            pl.BlockSpec((_BN, _EF), lambda i: (i, 0)),
            pl.BlockSpec((_BN, 1), lambda i: (i, 0)),
            pl.BlockSpec((_BN, 1), lambda i: (i, 0)),
            _rep((_NF, _H)),
            _rep((_EF, _H)),
            _rep((1, _H)),
            _rep((_H, _NF)),
            _rep((1, _NF)),
        ],
        out_specs=pl.BlockSpec((_BN, _NF), lambda i: (i, 0)),
        out_shape=jax.ShapeDtypeStruct((_N, _NF), jnp.float32),
    )(x, accp[0], accp[1],
      degp[0].reshape(_NP, 1), degp[1].reshape(_NP, 1),
      Wn1[0:_NF], Wn1[_NF:], bn1.reshape(1, _H), Wn2, bn2.reshape(1, _NF))

    return (edge_embedding, node_embeddings)


# speedup vs baseline: 8.6818x; 1.6347x over previous
import jax
import jax.numpy as jnp
from jax import lax
from jax.experimental import pallas as pl
from jax.experimental.pallas import tpu as pltpu
from jax.experimental.pallas import tpu_sc as plsc

_N = 100000
_E = 3200000
_NF = 16
_EF = 16
_H = 64

_NC = 2
_NS = 16
_NW = _NC * _NS
_NP = 100096
_TPW = _NP // _NS

_BE = 5120
_GS = _BE // 8
_NB = _E // _BE
_PR = _E // 8

_IB = 128
_CH = 640
_KI = _CH // _IB
_NCH = _E // _CH
_R2 = _E // _IB

_mesh = plsc.VectorSubcoreMesh(
    core_axis_name="c", subcore_axis_name="s", num_cores=_NC, num_subcores=_NS
)


def _packed_dst(ref, ch):
    e0 = ch * _CH
    i = e0 // _BE
    w = e0 % _BE
    g = w // _GS
    r0 = w % _GS
    return ref.at[pl.ds(i * _GS + r0, _CH), pl.ds(g * 16, 16)]


def _gather_body(x_hbm, row2_hbm, col2_hbm, xr8_hbm, xc8_hbm, degp_hbm,
                 idxr_v, idxc_v, xr_v, xc_v, ones_v, zdeg_v, deg_sh, sem):
    c = lax.axis_index("c")
    s = lax.axis_index("s")
    wid = s * _NC + c
    nb = (_NCH - wid + _NW - 1) // _NW

    def zfill_deg(i, carry):
        zdeg_v[pl.ds(i * 16, 16)] = jnp.zeros((16,), jnp.float32)
        return carry

    lax.fori_loop(0, 368 // 16, zfill_deg, 0)
    for i in range(_IB // 16):
        ones_v[pl.ds(i * 16, 16)] = jnp.ones((16,), jnp.float32)

    def zero_deg(z, carry):
        pltpu.sync_copy(zdeg_v, deg_sh.at[pl.ds(s * _TPW + z * 368, 368)])
        return carry

    lax.fori_loop(0, _TPW // 368, zero_deg, 0)
    plsc.subcore_barrier()

    def step(t, carry):
        ch = wid + t * _NW
        pltpu.sync_copy(row2_hbm.at[pl.ds(ch * _KI, _KI), :], idxr_v)
        pltpu.sync_copy(col2_hbm.at[pl.ds(ch * _KI, _KI), :], idxc_v)
        descs = []
        for j in range(_KI):
            descs.append(pltpu.async_copy(
                x_hbm.at[idxr_v.at[j]], xr_v.at[pl.ds(j * _IB, _IB), :], sem))
            descs.append(pltpu.async_copy(
                x_hbm.at[idxc_v.at[j]], xc_v.at[pl.ds(j * _IB, _IB), :], sem))
        for d in descs:
            d.wait()
        pltpu.sync_copy(xr_v, _packed_dst(xr8_hbm, ch))
        pltpu.sync_copy(xc_v, _packed_dst(xc8_hbm, ch))
        for j in range(_KI):
            pltpu.sync_copy(ones_v, deg_sh.at[idxr_v.at[j]], add=True)
        return carry

    lax.fori_loop(0, nb, step, 0)
    plsc.subcore_barrier()
    pltpu.sync_copy(deg_sh.at[pl.ds(s * _TPW, _TPW)],
                    degp_hbm.at[c, pl.ds(s * _TPW, _TPW)])


_gather_call = pl.kernel(
    _gather_body,
    out_type=(
        jax.ShapeDtypeStruct((_PR, 128), jnp.float32),
        jax.ShapeDtypeStruct((_PR, 128), jnp.float32),
        jax.ShapeDtypeStruct((_NC, _NP), jnp.float32),
    ),
    mesh=_mesh,
    scratch_types=[
        pltpu.VMEM((_KI, _IB), jnp.int32),
        pltpu.VMEM((_KI, _IB), jnp.int32),
        pltpu.VMEM((_CH, _NF), jnp.float32),
        pltpu.VMEM((_CH, _NF), jnp.float32),
        pltpu.VMEM((_IB,), jnp.float32),
        pltpu.VMEM((368,), jnp.float32),
        pltpu.VMEM_SHARED((_NP,), jnp.float32),
        pltpu.SemaphoreType.DMA,
    ],
    compiler_params=pltpu.CompilerParams(use_tc_tiling_on_sc=False),
)


def _scatter_body(row2_hbm, emb8_hbm, accp_hbm,
                  idxr_v, emb_v, zrow_v, acc_sh):
    c = lax.axis_index("c")
    s = lax.axis_index("s")
    wid = s * _NC + c

    def zfill_row(i, carry):
        zrow_v[i] = jnp.zeros((16,), jnp.float32)
        return carry

    lax.fori_loop(0, 368, zfill_row, 0)

    def zero_sh(z, carry):
        pltpu.sync_copy(zrow_v, acc_sh.at[pl.ds(s * _TPW + z * 368, 368), :])
        return carry

    lax.fori_loop(0, _TPW // 368, zero_sh, 0)
    plsc.subcore_barrier()

    def step(t, carry):
        ch = wid + t * _NW
        pltpu.sync_copy(row2_hbm.at[pl.ds(ch * _KI, _KI), :], idxr_v)
        pltpu.sync_copy(_packed_dst(emb8_hbm, ch), emb_v)
        for j in range(_KI):
            pltpu.sync_copy(emb_v.at[pl.ds(j * _IB, _IB), :],
                            acc_sh.at[idxr_v.at[j]], add=True)
        return carry

    nb = (_NCH - wid + _NW - 1) // _NW
    lax.fori_loop(0, nb, step, 0)
    plsc.subcore_barrier()

    pltpu.sync_copy(acc_sh.at[pl.ds(s * _TPW, _TPW), :],
                    accp_hbm.at[c, pl.ds(s * _TPW, _TPW), :])


_scatter_call = pl.kernel(
    _scatter_body,
    out_type=jax.ShapeDtypeStruct((_NC, _NP, _EF), jnp.float32),
    mesh=_mesh,
    scratch_types=[
        pltpu.VMEM((_KI, _IB), jnp.int32),
        pltpu.VMEM((_CH, _EF), jnp.float32),
        pltpu.VMEM((368, 16), jnp.float32),
        pltpu.VMEM_SHARED((_NP, _EF), jnp.float32),
    ],
    compiler_params=pltpu.CompilerParams(use_tc_tiling_on_sc=False),
)


def _edge_mlp_body(xr8_ref, xc8_ref, ea_ref, w1a_ref, w1b_ref, w1c_ref,
                   b1_ref, w2_ref, b2_ref, out_ref, out8_ref):
    for g in range(8):
        xr_g = xr8_ref[:, pl.ds(g * 16, 16)]
        xc_g = xc8_ref[:, pl.ds(g * 16, 16)]
        ea_g = ea_ref[pl.ds(g * _GS, _GS), :]
        h = (jnp.dot(xr_g, w1a_ref[...], preferred_element_type=jnp.float32)
             + jnp.dot(xc_g, w1b_ref[...], preferred_element_type=jnp.float32)
             + jnp.dot(ea_g, w1c_ref[...], preferred_element_type=jnp.float32)
             + b1_ref[...])
        h = jnp.maximum(h, 0.0)
        emb_g = (jnp.dot(h, w2_ref[...], preferred_element_type=jnp.float32)
                 + b2_ref[...])
        out_ref[pl.ds(g * _GS, _GS), :] = emb_g
        out8_ref[:, pl.ds(g * 16, 16)] = emb_g


_BN = 5000


def _node_mlp_body(x_ref, p0_ref, p1_ref, d0_ref, d1_ref,
                   w1a_ref, w1b_ref, b1_ref, w2_ref, b2_ref, out_ref):
    deg = d0_ref[...] + d1_ref[...]
    inv = 1.0 / jnp.maximum(deg, 1.0)
    agg = (p0_ref[...] + p1_ref[...]) * inv
    h = (jnp.dot(x_ref[...], w1a_ref[...], preferred_element_type=jnp.float32)
         + jnp.dot(agg, w1b_ref[...], preferred_element_type=jnp.float32)
         + b1_ref[...])
    h = jnp.maximum(h, 0.0)
    out_ref[...] = (jnp.dot(h, w2_ref[...], preferred_element_type=jnp.float32)
                    + b2_ref[...])


def _rep(shape):
    return pl.BlockSpec(shape, lambda i: tuple(0 for _ in shape))


def kernel(x, edge_index, edge_attr, We1, be1, We2, be2, Wn1, bn1, Wn2, bn2):
    x = x.astype(jnp.float32)
    edge_attr = edge_attr.astype(jnp.float32)
    row2 = edge_index[0].reshape(_R2, _IB)
    col2 = edge_index[1].reshape(_R2, _IB)

    xr8, xc8, degp = _gather_call(x, row2, col2)

    edge_embedding, emb8 = pl.pallas_call(
        _edge_mlp_body,
        grid=(_NB,),
        in_specs=[
            pl.BlockSpec((_GS, 128), lambda i: (i, 0)),
            pl.BlockSpec((_GS, 128), lambda i: (i, 0)),
            pl.BlockSpec((_BE, _EF), lambda i: (i, 0)),
            _rep((_NF, _H)),
            _rep((_NF, _H)),
            _rep((_EF, _H)),
            _rep((1, _H)),
            _rep((_H, _EF)),
            _rep((1, _EF)),
        ],
        out_specs=[
            pl.BlockSpec((_BE, _EF), lambda i: (i, 0)),
            pl.BlockSpec((_GS, 128), lambda i: (i, 0)),
        ],
        out_shape=[
            jax.ShapeDtypeStruct((_E, _EF), jnp.float32),
            jax.ShapeDtypeStruct((_PR, 128), jnp.float32),
        ],
    )(xr8, xc8, edge_attr,
      We1[0:_NF], We1[_NF:2 * _NF], We1[2 * _NF:],
      be1.reshape(1, _H), We2, be2.reshape(1, _EF))

    accp = _scatter_call(row2, emb8)

    node_embeddings = pl.pallas_call(
        _node_mlp_body,
        grid=(_N // _BN,),
        in_specs=[
            pl.BlockSpec((_BN, _NF), lambda i: (i, 0)),
            pl.BlockSpec((_BN, _EF), lambda i: (i, 0)),
            pl.BlockSpec((_BN, _EF), lambda i: (i, 0)),
            pl.BlockSpec((_BN, 1), lambda i: (i, 0)),
            pl.BlockSpec((_BN, 1), lambda i: (i, 0)),
            _rep((_NF, _H)),
            _rep((_EF, _H)),
            _rep((1, _H)),
            _rep((_H, _NF)),
            _rep((1, _NF)),
        ],
        out_specs=pl.BlockSpec((_BN, _NF), lambda i: (i, 0)),
        out_shape=jax.ShapeDtypeStruct((_N, _NF), jnp.float32),
    )(x, accp[0], accp[1],
      degp[0].reshape(_NP, 1), degp[1].reshape(_NP, 1),
      Wn1[0:_NF], Wn1[_NF:], bn1.reshape(1, _H), Wn2, bn2.reshape(1, _NF))

    return (edge_embedding, node_embeddings)


# speedup vs baseline: 9.5816x; 1.1036x over previous
import jax
import jax.numpy as jnp
from jax import lax
from jax.experimental import pallas as pl
from jax.experimental.pallas import tpu as pltpu
from jax.experimental.pallas import tpu_sc as plsc

_N = 100000
_E = 3200000
_NF = 16
_EF = 16
_H = 64

_NC = 2
_NS = 16
_NW = _NC * _NS
_NP = 100096
_TPW = _NP // _NS

_BE = 5120
_GS = _BE // 8
_NB = _E // _BE
_PR = _E // 8

_IB = 128
_CH = 640
_KI = _CH // _IB
_NCH = _E // _CH
_R2 = _E // _IB

_mesh = plsc.VectorSubcoreMesh(
    core_axis_name="c", subcore_axis_name="s", num_cores=_NC, num_subcores=_NS
)


def _packed_dst(ref, ch):
    e0 = ch * _CH
    i = e0 // _BE
    w = e0 % _BE
    g = w // _GS
    r0 = w % _GS
    return ref.at[pl.ds(i * _GS + r0, _CH), pl.ds(g * 16, 16)]


def _gather_body(x_hbm, row2_hbm, col2_hbm, xr8_hbm, xc8_hbm, degp_hbm,
                 idxr_v, idxc_v, xr_v, xc_v,
                 ones_v, zdeg_v, deg_sh, sia):
    c = lax.axis_index("c")
    s = lax.axis_index("s")
    wid = s * _NC + c
    nb = (_NCH - wid + _NW - 1) // _NW

    def zfill_deg(i, carry):
        zdeg_v[pl.ds(i * 16, 16)] = jnp.zeros((16,), jnp.float32)
        return carry

    lax.fori_loop(0, 368 // 16, zfill_deg, 0)
    for i in range(_IB // 16):
        ones_v[pl.ds(i * 16, 16)] = jnp.ones((16,), jnp.float32)

    def zero_deg(z, carry):
        pltpu.sync_copy(zdeg_v, deg_sh.at[pl.ds(s * _TPW + z * 368, 368)])
        return carry

    lax.fori_loop(0, _TPW // 368, zero_deg, 0)
    plsc.subcore_barrier()

    def step(t, carry):
        ch = wid + t * _NW
        pltpu.sync_copy(row2_hbm.at[pl.ds(ch * _KI, _KI), :], idxr_v)
        pltpu.sync_copy(col2_hbm.at[pl.ds(ch * _KI, _KI), :], idxc_v)
        descs = []
        for j in range(_KI):
            descs.append(pltpu.async_copy(
                x_hbm.at[idxr_v.at[j]], xr_v.at[pl.ds(j * _IB, _IB), :], sia))
            descs.append(pltpu.async_copy(
                x_hbm.at[idxc_v.at[j]], xc_v.at[pl.ds(j * _IB, _IB), :], sia))
        for d in descs:
            d.wait()
        pltpu.sync_copy(xr_v, _packed_dst(xr8_hbm, ch))
        pltpu.sync_copy(xc_v, _packed_dst(xc8_hbm, ch))
        for j in range(_KI):
            pltpu.sync_copy(ones_v, deg_sh.at[idxr_v.at[j]], add=True)
        return carry

    lax.fori_loop(0, nb, step, 0)
    plsc.subcore_barrier()
    pltpu.sync_copy(deg_sh.at[pl.ds(s * _TPW, _TPW)],
                    degp_hbm.at[c, pl.ds(s * _TPW, _TPW)])


_gather_call = pl.kernel(
    _gather_body,
    out_type=(
        jax.ShapeDtypeStruct((_PR, 128), jnp.float32),
        jax.ShapeDtypeStruct((_PR, 128), jnp.float32),
        jax.ShapeDtypeStruct((_NC, _NP), jnp.float32),
    ),
    mesh=_mesh,
    scratch_types=[
        pltpu.VMEM((_KI, _IB), jnp.int32),
        pltpu.VMEM((_KI, _IB), jnp.int32),
        pltpu.VMEM((_CH, _NF), jnp.float32),
        pltpu.VMEM((_CH, _NF), jnp.float32),
        pltpu.VMEM((_IB,), jnp.float32),
        pltpu.VMEM((368,), jnp.float32),
        pltpu.VMEM_SHARED((_NP,), jnp.float32),
        pltpu.SemaphoreType.DMA,
    ],
    compiler_params=pltpu.CompilerParams(use_tc_tiling_on_sc=False),
)


def _scatter_body(row2_hbm, emb8_hbm, accp_hbm,
                  idxr_v, emb_v, zrow_v, acc_sh):
    c = lax.axis_index("c")
    s = lax.axis_index("s")
    wid = s * _NC + c

    def zfill_row(i, carry):
        zrow_v[i] = jnp.zeros((16,), jnp.float32)
        return carry

    lax.fori_loop(0, 368, zfill_row, 0)

    def zero_sh(z, carry):
        pltpu.sync_copy(zrow_v, acc_sh.at[pl.ds(s * _TPW + z * 368, 368), :])
        return carry

    lax.fori_loop(0, _TPW // 368, zero_sh, 0)
    plsc.subcore_barrier()

    def step(t, carry):
        ch = wid + t * _NW
        pltpu.sync_copy(row2_hbm.at[pl.ds(ch * _KI, _KI), :], idxr_v)
        pltpu.sync_copy(_packed_dst(emb8_hbm, ch), emb_v)
        for j in range(_KI):
            pltpu.sync_copy(emb_v.at[pl.ds(j * _IB, _IB), :],
                            acc_sh.at[idxr_v.at[j]], add=True)
        return carry

    nb = (_NCH - wid + _NW - 1) // _NW
    lax.fori_loop(0, nb, step, 0)
    plsc.subcore_barrier()

    pltpu.sync_copy(acc_sh.at[pl.ds(s * _TPW, _TPW), :],
                    accp_hbm.at[c, pl.ds(s * _TPW, _TPW), :])


_scatter_call = pl.kernel(
    _scatter_body,
    out_type=jax.ShapeDtypeStruct((_NC, _NP, _EF), jnp.float32),
    mesh=_mesh,
    scratch_types=[
        pltpu.VMEM((_KI, _IB), jnp.int32),
        pltpu.VMEM((_CH, _EF), jnp.float32),
        pltpu.VMEM((368, 16), jnp.float32),
        pltpu.VMEM_SHARED((_NP, _EF), jnp.float32),
    ],
    compiler_params=pltpu.CompilerParams(use_tc_tiling_on_sc=False),
)


def _edge_mlp_body(xr8_ref, xc8_ref, ea_ref, w1a_ref, w1b_ref, w1c_ref,
                   b1_ref, w2_ref, b2_ref, out_ref, out8_ref):
    eah = (jnp.dot(ea_ref[...], w1c_ref[...], preferred_element_type=jnp.float32)
           + b1_ref[...])
    for g in range(8):
        xr_g = xr8_ref[:, pl.ds(g * 16, 16)]
        xc_g = xc8_ref[:, pl.ds(g * 16, 16)]
        h = (jnp.dot(xr_g, w1a_ref[...], preferred_element_type=jnp.float32)
             + jnp.dot(xc_g, w1b_ref[...], preferred_element_type=jnp.float32)
             + eah[g * _GS:(g + 1) * _GS, :])
        h = jnp.maximum(h, 0.0)
        emb_g = (jnp.dot(h, w2_ref[...], preferred_element_type=jnp.float32)
                 + b2_ref[...])
        out_ref[pl.ds(g * _GS, _GS), :] = emb_g
        out8_ref[:, pl.ds(g * 16, 16)] = emb_g


_BN = 5000


def _node_mlp_body(x_ref, p0_ref, p1_ref, d0_ref, d1_ref,
                   w1a_ref, w1b_ref, b1_ref, w2_ref, b2_ref, out_ref):
    deg = d0_ref[...] + d1_ref[...]
    inv = 1.0 / jnp.maximum(deg, 1.0)
    agg = (p0_ref[...] + p1_ref[...]) * inv
    h = (jnp.dot(x_ref[...], w1a_ref[...], preferred_element_type=jnp.float32)
         + jnp.dot(agg, w1b_ref[...], preferred_element_type=jnp.float32)
         + b1_ref[...])
    h = jnp.maximum(h, 0.0)
    out_ref[...] = (jnp.dot(h, w2_ref[...], preferred_element_type=jnp.float32)
                    + b2_ref[...])


def _rep(shape):
    return pl.BlockSpec(shape, lambda i: tuple(0 for _ in shape))


def kernel(x, edge_index, edge_attr, We1, be1, We2, be2, Wn1, bn1, Wn2, bn2):
    x = x.astype(jnp.float32)
    edge_attr = edge_attr.astype(jnp.float32)
    row2 = edge_index[0].reshape(_R2, _IB)
    col2 = edge_index[1].reshape(_R2, _IB)

    xr8, xc8, degp = _gather_call(x, row2, col2)

    edge_embedding, emb8 = pl.pallas_call(
        _edge_mlp_body,
        grid=(_NB,),
        in_specs=[
            pl.BlockSpec((_GS, 128), lambda i: (i, 0)),
            pl.BlockSpec((_GS, 128), lambda i: (i, 0)),
            pl.BlockSpec((_BE, _EF), lambda i: (i, 0)),
            _rep((_NF, _H)),
            _rep((_NF, _H)),
            _rep((_EF, _H)),
            _rep((1, _H)),
            _rep((_H, _EF)),
            _rep((1, _EF)),
        ],
        out_specs=[
            pl.BlockSpec((_BE, _EF), lambda i: (i, 0)),
            pl.BlockSpec((_GS, 128), lambda i: (i, 0)),
        ],
        out_shape=[
            jax.ShapeDtypeStruct((_E, _EF), jnp.float32),
            jax.ShapeDtypeStruct((_PR, 128), jnp.float32),
        ],
        compiler_params=pltpu.CompilerParams(vmem_limit_bytes=110 * 1024 * 1024),
    )(xr8, xc8, edge_attr,
      We1[0:_NF], We1[_NF:2 * _NF], We1[2 * _NF:],
      be1.reshape(1, _H), We2, be2.reshape(1, _EF))

    accp = _scatter_call(row2, emb8)

    node_embeddings = pl.pallas_call(
        _node_mlp_body,
        grid=(_N // _BN,),
        in_specs=[
            pl.BlockSpec((_BN, _NF), lambda i: (i, 0)),
            pl.BlockSpec((_BN, _EF), lambda i: (i, 0)),
            pl.BlockSpec((_BN, _EF), lambda i: (i, 0)),
            pl.BlockSpec((_BN, 1), lambda i: (i, 0)),
            pl.BlockSpec((_BN, 1), lambda i: (i, 0)),
            _rep((_NF, _H)),
            _rep((_EF, _H)),
            _rep((1, _H)),
            _rep((_H, _NF)),
            _rep((1, _NF)),
        ],
        out_specs=pl.BlockSpec((_BN, _NF), lambda i: (i, 0)),
        out_shape=jax.ShapeDtypeStruct((_N, _NF), jnp.float32),
    )(x, accp[0], accp[1],
      degp[0].reshape(_NP, 1), degp[1].reshape(_NP, 1),
      Wn1[0:_NF], Wn1[_NF:], bn1.reshape(1, _H), Wn2, bn2.reshape(1, _NF))

    return (edge_embedding, node_embeddings)


# speedup vs baseline: 9.7310x; 1.0156x over previous
import jax
import jax.numpy as jnp
from jax import lax
from jax.experimental import pallas as pl
from jax.experimental.pallas import tpu as pltpu
from jax.experimental.pallas import tpu_sc as plsc

_N = 100000
_E = 3200000
_NF = 16
_EF = 16
_H = 64

_NC = 2
_NS = 16
_NW = _NC * _NS
_NP = 100096
_TPW = _NP // _NS

_BE = 5120
_GS = _BE // 8
_NB = _E // _BE
_PR = _E // 8

_IB = 128
_CH = 640
_KI = _CH // _IB
_NCH = _E // _CH
_R2 = _E // _IB

_mesh = plsc.VectorSubcoreMesh(
    core_axis_name="c", subcore_axis_name="s", num_cores=_NC, num_subcores=_NS
)


def _packed_dst(ref, ch):
    e0 = ch * _CH
    i = e0 // _BE
    w = e0 % _BE
    g = w // _GS
    r0 = w % _GS
    return ref.at[pl.ds(i * _GS + r0, _CH), pl.ds(g * 16, 16)]


def _gather_body(x_hbm, row2_hbm, col2_hbm, xr8_hbm, xc8_hbm, degp_hbm,
                 idxr_v, idxc_v, xr_v, xc_v,
                 ones_v, zdeg_v, deg_sh, sia, sob, sod):
    c = lax.axis_index("c")
    s = lax.axis_index("s")
    wid = s * _NC + c
    nb = (_NCH - wid + _NW - 1) // _NW

    def zfill_deg(i, carry):
        zdeg_v[pl.ds(i * 16, 16)] = jnp.zeros((16,), jnp.float32)
        return carry

    lax.fori_loop(0, 368 // 16, zfill_deg, 0)
    for i in range(_IB // 16):
        ones_v[pl.ds(i * 16, 16)] = jnp.ones((16,), jnp.float32)

    def zero_deg(z, carry):
        pltpu.sync_copy(zdeg_v, deg_sh.at[pl.ds(s * _TPW + z * 368, 368)])
        return carry

    lax.fori_loop(0, _TPW // 368, zero_deg, 0)
    plsc.subcore_barrier()

    def step(t, carry):
        ch = wid + t * _NW
        pltpu.sync_copy(row2_hbm.at[pl.ds(ch * _KI, _KI), :], idxr_v)
        pltpu.sync_copy(col2_hbm.at[pl.ds(ch * _KI, _KI), :], idxc_v)
        descs = []
        for j in range(_KI):
            descs.append(pltpu.async_copy(
                x_hbm.at[idxr_v.at[j]], xr_v.at[pl.ds(j * _IB, _IB), :], sia))
            descs.append(pltpu.async_copy(
                x_hbm.at[idxc_v.at[j]], xc_v.at[pl.ds(j * _IB, _IB), :], sia))
        for d in descs:
            d.wait()
        tail = [pltpu.async_copy(xr_v, _packed_dst(xr8_hbm, ch), sob),
                pltpu.async_copy(xc_v, _packed_dst(xc8_hbm, ch), sob)]
        tail += [pltpu.async_copy(ones_v, deg_sh.at[idxr_v.at[j]], sod,
                                  add=True)
                 for j in range(_KI)]
        for d in tail:
            d.wait()
        return carry

    lax.fori_loop(0, nb, step, 0)
    plsc.subcore_barrier()
    pltpu.sync_copy(deg_sh.at[pl.ds(s * _TPW, _TPW)],
                    degp_hbm.at[c, pl.ds(s * _TPW, _TPW)])


_gather_call = pl.kernel(
    _gather_body,
    out_type=(
        jax.ShapeDtypeStruct((_PR, 128), jnp.float32),
        jax.ShapeDtypeStruct((_PR, 128), jnp.float32),
        jax.ShapeDtypeStruct((_NC, _NP), jnp.float32),
    ),
    mesh=_mesh,
    scratch_types=[
        pltpu.VMEM((_KI, _IB), jnp.int32),
        pltpu.VMEM((_KI, _IB), jnp.int32),
        pltpu.VMEM((_CH, _NF), jnp.float32),
        pltpu.VMEM((_CH, _NF), jnp.float32),
        pltpu.VMEM((_IB,), jnp.float32),
        pltpu.VMEM((368,), jnp.float32),
        pltpu.VMEM_SHARED((_NP,), jnp.float32),
        pltpu.SemaphoreType.DMA,
        pltpu.SemaphoreType.DMA,
        pltpu.SemaphoreType.DMA,
    ],
    compiler_params=pltpu.CompilerParams(use_tc_tiling_on_sc=False),
)


def _scatter_body(row2_hbm, emb8_hbm, accp_hbm,
                  idxr_v, emb_v, zrow_v, acc_sh, ssa):
    c = lax.axis_index("c")
    s = lax.axis_index("s")
    wid = s * _NC + c

    def zfill_row(i, carry):
        zrow_v[i] = jnp.zeros((16,), jnp.float32)
        return carry

    lax.fori_loop(0, 368, zfill_row, 0)

    def zero_sh(z, carry):
        pltpu.sync_copy(zrow_v, acc_sh.at[pl.ds(s * _TPW + z * 368, 368), :])
        return carry

    lax.fori_loop(0, _TPW // 368, zero_sh, 0)
    plsc.subcore_barrier()

    def step(t, carry):
        ch = wid + t * _NW
        pltpu.sync_copy(row2_hbm.at[pl.ds(ch * _KI, _KI), :], idxr_v)
        pltpu.sync_copy(_packed_dst(emb8_hbm, ch), emb_v)
        adds = [pltpu.async_copy(emb_v.at[pl.ds(j * _IB, _IB), :],
                                 acc_sh.at[idxr_v.at[j]], ssa, add=True)
                for j in range(_KI)]
        for d in adds:
            d.wait()
        return carry

    nb = (_NCH - wid + _NW - 1) // _NW
    lax.fori_loop(0, nb, step, 0)
    plsc.subcore_barrier()

    pltpu.sync_copy(acc_sh.at[pl.ds(s * _TPW, _TPW), :],
                    accp_hbm.at[c, pl.ds(s * _TPW, _TPW), :])


_scatter_call = pl.kernel(
    _scatter_body,
    out_type=jax.ShapeDtypeStruct((_NC, _NP, _EF), jnp.float32),
    mesh=_mesh,
    scratch_types=[
        pltpu.VMEM((_KI, _IB), jnp.int32),
        pltpu.VMEM((_CH, _EF), jnp.float32),
        pltpu.VMEM((368, 16), jnp.float32),
        pltpu.VMEM_SHARED((_NP, _EF), jnp.float32),
        pltpu.SemaphoreType.DMA,
    ],
    compiler_params=pltpu.CompilerParams(use_tc_tiling_on_sc=False),
)


def _edge_mlp_body(xr8_ref, xc8_ref, ea_ref, w1a_ref, w1b_ref, w1c_ref,
                   b1_ref, w2_ref, b2_ref, out_ref, out8_ref):
    eah = (jnp.dot(ea_ref[...], w1c_ref[...], preferred_element_type=jnp.float32)
           + b1_ref[...])
    for g in range(8):
        xr_g = xr8_ref[:, pl.ds(g * 16, 16)]
        xc_g = xc8_ref[:, pl.ds(g * 16, 16)]
        h = (jnp.dot(xr_g, w1a_ref[...], preferred_element_type=jnp.float32)
             + jnp.dot(xc_g, w1b_ref[...], preferred_element_type=jnp.float32)
             + eah[g * _GS:(g + 1) * _GS, :])
        h = jnp.maximum(h, 0.0)
        emb_g = (jnp.dot(h, w2_ref[...], preferred_element_type=jnp.float32)
                 + b2_ref[...])
        out_ref[pl.ds(g * _GS, _GS), :] = emb_g
        out8_ref[:, pl.ds(g * 16, 16)] = emb_g


_BN = 5000


def _node_mlp_body(x_ref, p0_ref, p1_ref, d0_ref, d1_ref,
                   w1a_ref, w1b_ref, b1_ref, w2_ref, b2_ref, out_ref):
    deg = d0_ref[...] + d1_ref[...]
    inv = 1.0 / jnp.maximum(deg, 1.0)
    agg = (p0_ref[...] + p1_ref[...]) * inv
    h = (jnp.dot(x_ref[...], w1a_ref[...], preferred_element_type=jnp.float32)
         + jnp.dot(agg, w1b_ref[...], preferred_element_type=jnp.float32)
         + b1_ref[...])
    h = jnp.maximum(h, 0.0)
    out_ref[...] = (jnp.dot(h, w2_ref[...], preferred_element_type=jnp.float32)
                    + b2_ref[...])


def _rep(shape):
    return pl.BlockSpec(shape, lambda i: tuple(0 for _ in shape))


def kernel(x, edge_index, edge_attr, We1, be1, We2, be2, Wn1, bn1, Wn2, bn2):
    x = x.astype(jnp.float32)
    edge_attr = edge_attr.astype(jnp.float32)
    row2 = edge_index[0].reshape(_R2, _IB)
    col2 = edge_index[1].reshape(_R2, _IB)

    xr8, xc8, degp = _gather_call(x, row2, col2)

    edge_embedding, emb8 = pl.pallas_call(
        _edge_mlp_body,
        grid=(_NB,),
        in_specs=[
            pl.BlockSpec((_GS, 128), lambda i: (i, 0)),
            pl.BlockSpec((_GS, 128), lambda i: (i, 0)),
            pl.BlockSpec((_BE, _EF), lambda i: (i, 0)),
            _rep((_NF, _H)),
            _rep((_NF, _H)),
            _rep((_EF, _H)),
            _rep((1, _H)),
            _rep((_H, _EF)),
            _rep((1, _EF)),
        ],
        out_specs=[
            pl.BlockSpec((_BE, _EF), lambda i: (i, 0)),
            pl.BlockSpec((_GS, 128), lambda i: (i, 0)),
        ],
        out_shape=[
            jax.ShapeDtypeStruct((_E, _EF), jnp.float32),
            jax.ShapeDtypeStruct((_PR, 128), jnp.float32),
        ],
        compiler_params=pltpu.CompilerParams(vmem_limit_bytes=110 * 1024 * 1024),
    )(xr8, xc8, edge_attr,
      We1[0:_NF], We1[_NF:2 * _NF], We1[2 * _NF:],
      be1.reshape(1, _H), We2, be2.reshape(1, _EF))

    accp = _scatter_call(row2, emb8)

    node_embeddings = pl.pallas_call(
        _node_mlp_body,
        grid=(_N // _BN,),
        in_specs=[
            pl.BlockSpec((_BN, _NF), lambda i: (i, 0)),
            pl.BlockSpec((_BN, _EF), lambda i: (i, 0)),
            pl.BlockSpec((_BN, _EF), lambda i: (i, 0)),
            pl.BlockSpec((_BN, 1), lambda i: (i, 0)),
            pl.BlockSpec((_BN, 1), lambda i: (i, 0)),
            _rep((_NF, _H)),
            _rep((_EF, _H)),
            _rep((1, _H)),
            _rep((_H, _NF)),
            _rep((1, _NF)),
        ],
        out_specs=pl.BlockSpec((_BN, _NF), lambda i: (i, 0)),
        out_shape=jax.ShapeDtypeStruct((_N, _NF), jnp.float32),
    )(x, accp[0], accp[1],
      degp[0].reshape(_NP, 1), degp[1].reshape(_NP, 1),
      Wn1[0:_NF], Wn1[_NF:], bn1.reshape(1, _H), Wn2, bn2.reshape(1, _NF))

    return (edge_embedding, node_embeddings)
